# Initial kernel scaffold; baseline (speedup 1.0000x reference)
import jax
import jax.numpy as jnp
from jax.experimental import pallas as pl


def kernel(static_tensor, drug_indices, comorb_indices, drug_table, comorb_table, W, b):
    raise NotImplementedError("write your pallas kernel here")


# speedup vs baseline: 7.9939x; 7.9939x over previous
import functools

import jax
import jax.numpy as jnp
from jax import lax
from jax.experimental import pallas as pl
from jax.experimental.pallas import tpu as pltpu
from jax.experimental.pallas import tpu_sc as plsc

B = 4096
STATIC_DIM = 128
EMB = 64
HID = 128
L = 50

NC = 2
NS = 16
NW = NC * NS
BPW = B // NW
CHUNK = 8
NCHUNK = BPW // CHUNK
ROWS_PER_GATHER = CHUNK * L

_sc_mesh = plsc.VectorSubcoreMesh(core_axis_name="c", subcore_axis_name="s")


@functools.partial(
    pl.kernel,
    out_type=(
        jax.ShapeDtypeStruct((B, EMB), jnp.float32),
        jax.ShapeDtypeStruct((B, EMB), jnp.float32),
    ),
    mesh=_sc_mesh,
    scratch_types=[
        pltpu.VMEM((BPW * L,), jnp.int32),
        pltpu.VMEM((ROWS_PER_GATHER, EMB), jnp.float32),
        pltpu.VMEM((BPW, EMB), jnp.float32),
        pltpu.SemaphoreType.DMA,
    ],
    compiler_params=pltpu.CompilerParams(use_tc_tiling_on_sc=False),
)
def _emb_mean_sc(d_idx_hbm, c_idx_hbm, d_tab_hbm, c_tab_hbm,
                 d_out_hbm, c_out_hbm, idx_v, rows_v, out_v, sem):
    wid = lax.axis_index("s") * NC + lax.axis_index("c")
    ibase = wid * (BPW * L)
    obase = wid * BPW
    inv_l = jnp.float32(1.0 / L)

    def one_table(idx_hbm, tab_hbm, out_hbm):
        pltpu.sync_copy(idx_hbm.at[pl.ds(ibase, BPW * L)], idx_v)

        def chunk_body(c, carry):
            off = pl.multiple_of(c * ROWS_PER_GATHER, ROWS_PER_GATHER)
            pltpu.async_copy(
                tab_hbm.at[idx_v.at[pl.ds(off, ROWS_PER_GATHER)]],
                rows_v, sem).wait()

            def reduce_row(r, _):
                rb = r * L
                for q in range(EMB // 16):
                    sl = pl.ds(q * 16, 16)
                    acc = rows_v[rb, sl]
                    for j in range(1, L):
                        acc = acc + rows_v[rb + j, sl]
                    out_v[c * CHUNK + r, sl] = acc * inv_l
                return _

            lax.fori_loop(0, CHUNK, reduce_row, None)
            return carry

        lax.fori_loop(0, NCHUNK, chunk_body, None)
        pltpu.sync_copy(out_v, out_hbm.at[pl.ds(obase, BPW)])

    one_table(d_idx_hbm, d_tab_hbm, d_out_hbm)
    one_table(c_idx_hbm, c_tab_hbm, c_out_hbm)


def _linear_tc_body(x_ref, w_ref, b_ref, o_ref):
    o_ref[...] = (
        jnp.dot(x_ref[...], w_ref[...], preferred_element_type=jnp.float32)
        + b_ref[...]
    )


_ROWS_BLK = 512


def _linear_tc(x, w, b2d):
    return pl.pallas_call(
        _linear_tc_body,
        grid=(B // _ROWS_BLK,),
        in_specs=[
            pl.BlockSpec((_ROWS_BLK, STATIC_DIM), lambda i: (i, 0)),
            pl.BlockSpec((STATIC_DIM, HID), lambda i: (0, 0)),
            pl.BlockSpec((1, HID), lambda i: (0, 0)),
        ],
        out_specs=pl.BlockSpec((_ROWS_BLK, HID), lambda i: (i, 0)),
        out_shape=jax.ShapeDtypeStruct((B, HID), jnp.float32),
    )(x, w, b2d)


def kernel(static_tensor, drug_indices, comorb_indices, drug_table,
           comorb_table, W, b):
    d_idx = drug_indices.reshape(-1)
    c_idx = comorb_indices.reshape(-1)
    d_mean, c_mean = _emb_mean_sc(d_idx, c_idx, drug_table, comorb_table)
    static_repr = _linear_tc(static_tensor, W, b.reshape(1, HID))
    return jnp.concatenate([static_repr, d_mean, c_mean], axis=1)


# speedup vs baseline: 10.3044x; 1.2890x over previous
import functools

import jax
import jax.numpy as jnp
from jax import lax
from jax.experimental import pallas as pl
from jax.experimental.pallas import tpu as pltpu
from jax.experimental.pallas import tpu_sc as plsc

B = 4096
STATIC_DIM = 128
EMB = 64
HID = 128
L = 50

NC = 2
NS = 16
NW = NC * NS
BPW = B // NW
CHUNK = 8
NCHUNK = BPW // CHUNK
ROWS_PER_GATHER = CHUNK * L

_sc_mesh = plsc.VectorSubcoreMesh(core_axis_name="c", subcore_axis_name="s")


@functools.partial(
    pl.kernel,
    out_type=(
        jax.ShapeDtypeStruct((B, EMB), jnp.float32),
        jax.ShapeDtypeStruct((B, EMB), jnp.float32),
    ),
    mesh=_sc_mesh,
    scratch_types=[
        pltpu.VMEM((BPW * L,), jnp.int32),
        pltpu.VMEM((BPW * L,), jnp.int32),
        pltpu.VMEM((ROWS_PER_GATHER, EMB), jnp.float32),
        pltpu.VMEM((ROWS_PER_GATHER, EMB), jnp.float32),
        pltpu.VMEM((BPW, EMB), jnp.float32),
        pltpu.SemaphoreType.DMA,
        pltpu.SemaphoreType.DMA,
    ],
    compiler_params=pltpu.CompilerParams(use_tc_tiling_on_sc=False),
)
def _emb_mean_sc(d_idx_hbm, c_idx_hbm, d_tab_hbm, c_tab_hbm,
                 d_out_hbm, c_out_hbm, d_idx_v, c_idx_v, rows0_v, rows1_v,
                 out_v, sem0, sem1):
    wid = lax.axis_index("s") * NC + lax.axis_index("c")
    ibase = wid * (BPW * L)
    obase = wid * BPW
    inv_l = jnp.float32(1.0 / L)

    pltpu.sync_copy(d_idx_hbm.at[pl.ds(ibase, BPW * L)], d_idx_v)
    pltpu.sync_copy(c_idx_hbm.at[pl.ds(ibase, BPW * L)], c_idx_v)

    def one_table(idx_v, tab_hbm, out_hbm):
        def gather(c, buf, sem):
            off = pl.multiple_of(c * ROWS_PER_GATHER, ROWS_PER_GATHER)
            return pltpu.make_async_copy(
                tab_hbm.at[idx_v.at[pl.ds(off, ROWS_PER_GATHER)]], buf, sem)

        def reduce_chunk(c, buf):
            def reduce_row(r, _):
                rb = r * L
                for q in range(EMB // 16):
                    sl = pl.ds(q * 16, 16)
                    a0 = buf[rb, sl]
                    a1 = buf[rb + 1, sl]
                    for j in range(2, L, 2):
                        a0 = a0 + buf[rb + j, sl]
                        a1 = a1 + buf[rb + j + 1, sl]
                    out_v[c * CHUNK + r, sl] = (a0 + a1) * inv_l
                return _

            lax.fori_loop(0, CHUNK, reduce_row, None)

        gather(0, rows0_v, sem0).start()

        def pair_body(cc, carry):
            c0 = cc * 2
            gather(c0, rows0_v, sem0).wait()
            gather(c0 + 1, rows1_v, sem1).start()
            reduce_chunk(c0, rows0_v)
            gather(c0 + 1, rows1_v, sem1).wait()

            @pl.when(cc < NCHUNK // 2 - 1)
            def _():
                gather(c0 + 2, rows0_v, sem0).start()

            reduce_chunk(c0 + 1, rows1_v)
            return carry

        lax.fori_loop(0, NCHUNK // 2, pair_body, None)
        pltpu.sync_copy(out_v, out_hbm.at[pl.ds(obase, BPW)])

    one_table(d_idx_v, d_tab_hbm, d_out_hbm)
    one_table(c_idx_v, c_tab_hbm, c_out_hbm)


def _linear_tc_body(x_ref, w_ref, b_ref, o_ref):
    o_ref[...] = (
        jnp.dot(x_ref[...], w_ref[...], preferred_element_type=jnp.float32)
        + b_ref[...]
    )


_ROWS_BLK = 512


def _linear_tc(x, w, b2d):
    return pl.pallas_call(
        _linear_tc_body,
        grid=(B // _ROWS_BLK,),
        in_specs=[
            pl.BlockSpec((_ROWS_BLK, STATIC_DIM), lambda i: (i, 0)),
            pl.BlockSpec((STATIC_DIM, HID), lambda i: (0, 0)),
            pl.BlockSpec((1, HID), lambda i: (0, 0)),
        ],
        out_specs=pl.BlockSpec((_ROWS_BLK, HID), lambda i: (i, 0)),
        out_shape=jax.ShapeDtypeStruct((B, HID), jnp.float32),
    )(x, w, b2d)


def kernel(static_tensor, drug_indices, comorb_indices, drug_table,
           comorb_table, W, b):
    d_idx = drug_indices.reshape(-1)
    c_idx = comorb_indices.reshape(-1)
    d_mean, c_mean = _emb_mean_sc(d_idx, c_idx, drug_table, comorb_table)
    static_repr = _linear_tc(static_tensor, W, b.reshape(1, HID))
    return jnp.concatenate([static_repr, d_mean, c_mean], axis=1)
